# baseline (speedup 1.0000x reference)
import numpy as np
import jax
import jax.numpy as jnp
from jax.experimental import pallas as pl
from jax.experimental.pallas import tpu as pltpu

_THRESH = 0.6
_MIN_KEPT = 200000
_FACTOR = 8
_IGNORE = -1

_N, _C, _H, _W = 4, 19, 512, 512
_OH, _OW = 64, 64
_NDS = _N * _OH * _OW
_K = min(_NDS, _MIN_KEPT // (_FACTOR * _FACTOR)) - 1
_R = 128

_INTERPRET = False


def _grid_coords(size, out):
    c = (np.arange(out) * (size - 1)).astype(np.float32) / np.float32(out - 1)
    lo = np.floor(c).astype(np.int32)
    hi = np.minimum(lo + 1, size - 1).astype(np.int32)
    frac = (c - lo.astype(np.float32)).astype(np.float32)
    near = np.clip(np.floor(c + 0.5).astype(np.int32), 0, size - 1)
    return lo, hi, frac, near


_H0, _H1, _FH, _IH = _grid_coords(_H, _OH)
_W0, _W1, _FW, _IW = _grid_coords(_W, _OW)

_pp = np.arange(_NDS)
_FHM = _FH[(_pp // _OW) % _OH].reshape(_R, _R)
_FWM = _FW[_pp % _OW].reshape(_R, _R)


_WMAP = np.zeros((1, _W), np.float32)
_WMAP[0, _W0] += (np.float32(1.0) - _FW)
_WMAP[0, _W1] += _FW
_PSEL = np.zeros((_W, _OW), np.float32)
_PSEL[_W0, np.arange(_OW)] = 1.0
_PSEL[_W1, np.arange(_OW)] = 1.0
_JMAP = np.zeros(_W, np.int32)
_VALID = np.zeros(_W, bool)
_JMAP[_W0] = np.arange(_OW)
_VALID[_W0] = True
_JMAP[_W1] = np.arange(_OW)
_VALID[_W1] = True


def _threshold_kernel(x0_ref, x1_ref, lmap_ref, wmap_ref, psel_ref,
                      out, acc_ref):
    i = pl.program_id(0)

    ch = (i * (_H - 1)).astype(jnp.float32) / jnp.float32(_OH - 1)
    fh = ch - jnp.floor(ch)

    lrow = lmap_ref[:, pl.ds(i, 1), :]

    def row_pred(xr):
        x = xr[...].reshape(_N, _C, _W)
        m = jnp.max(x, axis=1, keepdims=True)
        e = jnp.exp(x - m)
        s = jnp.sum(e, axis=1, keepdims=True)
        cio = jax.lax.broadcasted_iota(jnp.int32, (_N, _C, _W), 1)
        t = jnp.sum(jnp.where(cio == lrow, e, 0.0), axis=1, keepdims=True)
        return (t / s).reshape(_N, _W)

    p0 = row_pred(x0_ref)
    p1 = row_pred(x1_ref)
    r = p0 * (1.0 - fh) + p1 * fh
    rw = r * wmap_ref[...]
    acc_ref[pl.ds(i, 1), :] = rw.reshape(1, _N * _W)

    @pl.when(i == _OH - 1)
    def _finish():
        g = acc_ref[...].reshape(_OH * _N, _W)
        pred = jax.lax.dot_general(
            g, psel_ref[...],
            (((1,), (0,)), ((), ())),
            preferred_element_type=jnp.float32)
        v = jax.lax.bitcast_convert_type(pred, jnp.int32)

        def body(_, carry):
            lo_b, hi_b = carry
            mid = lo_b + (hi_b - lo_b) // 2
            cnt = jnp.sum((v <= mid).astype(jnp.int32))
            ge = cnt >= (_K + 1)
            return (jnp.where(ge, lo_b, mid + 1), jnp.where(ge, mid, hi_b))

        lo_b, _hi = jax.lax.fori_loop(
            0, 31, body, (jnp.int32(0), jnp.int32(0x7F7FFFFF)))
        kth = jax.lax.bitcast_convert_type(lo_b, jnp.float32)
        out[0, 0] = jnp.where(kth > _THRESH, kth, jnp.float32(_THRESH))


def _mask_kernel(thr, x, lbl, out):
    t = thr[0, 0]
    l = lbl[0]
    m = x[0, 0]
    for c in range(1, _C):
        m = jnp.maximum(m, x[0, c])
    s = jnp.zeros_like(m)
    el = jnp.zeros_like(m)
    for c in range(_C):
        e = jnp.exp(x[0, c] - m)
        s = s + e
        el = jnp.where(l == c, e, el)
    pred = el / s
    keep = (l >= 0) & (pred <= t)
    out[0] = jnp.where(keep, l, _IGNORE)


_BH = 128


def kernel(predict, target):
    lbl32 = target.astype(jnp.int32)

    lbl_ds = lbl32[:, _IH][:, :, _IW]
    lmap = jnp.where(jnp.asarray(_VALID)[None, None, :],
                     lbl_ds[:, :, _JMAP], -1)

    predict5d = predict.reshape(_N, _C, _H, 1, _W)

    def _row0(i):
        return (0, 0, (i * (_H - 1)) // (_OH - 1), 0, 0)

    def _row1(i):
        return (0, 0, jnp.minimum((i * (_H - 1)) // (_OH - 1) + 1, _H - 1),
                0, 0)

    thr = pl.pallas_call(
        _threshold_kernel,
        grid=(_OH,),
        in_specs=[
            pl.BlockSpec((_N, _C, 1, 1, _W), _row0),
            pl.BlockSpec((_N, _C, 1, 1, _W), _row1),
            pl.BlockSpec((_N, _OH, _W), lambda i: (0, 0, 0)),
            pl.BlockSpec((1, _W), lambda i: (0, 0)),
            pl.BlockSpec((_W, _OW), lambda i: (0, 0)),
        ],
        out_shape=jax.ShapeDtypeStruct((1, 1), jnp.float32),
        out_specs=pl.BlockSpec(memory_space=pltpu.SMEM),
        scratch_shapes=[pltpu.VMEM((_OH, _N * _W), jnp.float32)],
        interpret=_INTERPRET,
    )(predict5d, predict5d, lmap, jnp.asarray(_WMAP), jnp.asarray(_PSEL))

    out = pl.pallas_call(
        _mask_kernel,
        grid=(_N, _H // _BH),
        in_specs=[
            pl.BlockSpec(memory_space=pltpu.SMEM),
            pl.BlockSpec((1, _C, _BH, _W), lambda n, h: (n, 0, h, 0)),
            pl.BlockSpec((1, _BH, _W), lambda n, h: (n, h, 0)),
        ],
        out_specs=pl.BlockSpec((1, _BH, _W), lambda n, h: (n, h, 0)),
        out_shape=jax.ShapeDtypeStruct((_N, _H, _W), jnp.int32),
        interpret=_INTERPRET,
    )(thr, predict, lbl32)

    return out.astype(jnp.int64)


# speedup vs baseline: 1.0125x; 1.0125x over previous
import numpy as np
import jax
import jax.numpy as jnp
from jax.experimental import pallas as pl
from jax.experimental.pallas import tpu as pltpu

_THRESH = 0.6
_MIN_KEPT = 200000
_FACTOR = 8
_IGNORE = -1

_N, _C, _H, _W = 4, 19, 512, 512
_OH, _OW = 64, 64
_NDS = _N * _OH * _OW
_K = min(_NDS, _MIN_KEPT // (_FACTOR * _FACTOR)) - 1
_R = 128

_INTERPRET = False


def _grid_coords(size, out):
    c = (np.arange(out) * (size - 1)).astype(np.float32) / np.float32(out - 1)
    lo = np.floor(c).astype(np.int32)
    hi = np.minimum(lo + 1, size - 1).astype(np.int32)
    frac = (c - lo.astype(np.float32)).astype(np.float32)
    near = np.clip(np.floor(c + 0.5).astype(np.int32), 0, size - 1)
    return lo, hi, frac, near


_H0, _H1, _FH, _IH = _grid_coords(_H, _OH)
_W0, _W1, _FW, _IW = _grid_coords(_W, _OW)

_pp = np.arange(_NDS)
_FHM = _FH[(_pp // _OW) % _OH].reshape(_R, _R)
_FWM = _FW[_pp % _OW].reshape(_R, _R)


_CI = np.empty(2 * _OW, np.int32)
_CI[0::2] = _W0
_CI[1::2] = _W1
_NC2 = 2 * _OW
_CSEL = np.zeros((_W, _NC2), np.float32)
_CSEL[_CI, np.arange(_NC2)] = 1.0
_W2 = np.empty((1, _NC2), np.float32)
_W2[0, 0::2] = np.float32(1.0) - _FW
_W2[0, 1::2] = _FW
_PAIR = np.zeros((_NC2, _OW), np.float32)
_PAIR[np.arange(0, _NC2, 2), np.arange(_OW)] = 1.0
_PAIR[np.arange(1, _NC2, 2), np.arange(_OW)] = 1.0


def _threshold_kernel(x0_ref, x1_ref, lrep_ref, csel_ref, pair_ref, w2_ref,
                      out, acc_ref):
    i = pl.program_id(0)

    ch = (i * (_H - 1)).astype(jnp.float32) / jnp.float32(_OH - 1)
    fh = ch - jnp.floor(ch)

    lrow = lrep_ref[:, pl.ds(i, 1), :]
    csel = csel_ref[...]

    def row_pred(xr):
        x = xr[...].reshape(_N * _C, _W)
        g = jax.lax.dot_general(
            x, csel, (((1,), (0,)), ((), ())),
            preferred_element_type=jnp.float32).reshape(_N, _C, _NC2)
        m = jnp.max(g, axis=1, keepdims=True)
        e = jnp.exp(g - m)
        s = jnp.sum(e, axis=1, keepdims=True)
        cio = jax.lax.broadcasted_iota(jnp.int32, (_N, _C, _NC2), 1)
        t = jnp.sum(jnp.where(cio == lrow, e, 0.0), axis=1, keepdims=True)
        return (t / s).reshape(_N, _NC2)

    p0 = row_pred(x0_ref)
    p1 = row_pred(x1_ref)
    r = p0 * (1.0 - fh) + p1 * fh
    rw = r * w2_ref[...]
    acc_ref[pl.ds(i, 1), :] = rw.reshape(1, _N * _NC2)

    @pl.when(i == _OH - 1)
    def _finish():
        g = acc_ref[...].reshape(_OH * _N, _NC2)
        pred = jax.lax.dot_general(
            g, pair_ref[...],
            (((1,), (0,)), ((), ())),
            preferred_element_type=jnp.float32)
        v = jax.lax.bitcast_convert_type(pred, jnp.int32)

        def body(_, carry):
            lo_b, hi_b = carry
            mid = lo_b + (hi_b - lo_b) // 2
            cnt = jnp.sum((v <= mid).astype(jnp.int32))
            ge = cnt >= (_K + 1)
            return (jnp.where(ge, lo_b, mid + 1), jnp.where(ge, mid, hi_b))

        lo_b, _hi = jax.lax.fori_loop(
            0, 31, body, (jnp.int32(0), jnp.int32(0x7F7FFFFF)))
        kth = jax.lax.bitcast_convert_type(lo_b, jnp.float32)
        out[0, 0] = jnp.where(kth > _THRESH, kth, jnp.float32(_THRESH))


def _mask_kernel(thr, x, lbl, out):
    t = thr[0, 0]
    l = lbl[0]
    m = x[0, 0]
    for c in range(1, _C):
        m = jnp.maximum(m, x[0, c])
    s = jnp.zeros_like(m)
    el = jnp.zeros_like(m)
    for c in range(_C):
        e = jnp.exp(x[0, c] - m)
        s = s + e
        el = jnp.where(l == c, e, el)
    pred = el / s
    keep = (l >= 0) & (pred <= t)
    out[0] = jnp.where(keep, l, _IGNORE)


_BH = 128


def kernel(predict, target):
    lbl32 = target.astype(jnp.int32)

    lbl_ds = lbl32[:, _IH][:, :, _IW]
    lrep = jnp.repeat(lbl_ds, 2, axis=2)

    predict5d = predict.reshape(_N, _C, _H, 1, _W)

    def _row0(i):
        return (0, 0, (i * (_H - 1)) // (_OH - 1), 0, 0)

    def _row1(i):
        return (0, 0, jnp.minimum((i * (_H - 1)) // (_OH - 1) + 1, _H - 1),
                0, 0)

    thr = pl.pallas_call(
        _threshold_kernel,
        grid=(_OH,),
        in_specs=[
            pl.BlockSpec((_N, _C, 1, 1, _W), _row0),
            pl.BlockSpec((_N, _C, 1, 1, _W), _row1),
            pl.BlockSpec((_N, _OH, _NC2), lambda i: (0, 0, 0)),
            pl.BlockSpec((_W, _NC2), lambda i: (0, 0)),
            pl.BlockSpec((_NC2, _OW), lambda i: (0, 0)),
            pl.BlockSpec((1, _NC2), lambda i: (0, 0)),
        ],
        out_shape=jax.ShapeDtypeStruct((1, 1), jnp.float32),
        out_specs=pl.BlockSpec(memory_space=pltpu.SMEM),
        scratch_shapes=[pltpu.VMEM((_OH, _N * _NC2), jnp.float32)],
        interpret=_INTERPRET,
    )(predict5d, predict5d, lrep, jnp.asarray(_CSEL), jnp.asarray(_PAIR),
      jnp.asarray(_W2))

    out = pl.pallas_call(
        _mask_kernel,
        grid=(_N, _H // _BH),
        in_specs=[
            pl.BlockSpec(memory_space=pltpu.SMEM),
            pl.BlockSpec((1, _C, _BH, _W), lambda n, h: (n, 0, h, 0)),
            pl.BlockSpec((1, _BH, _W), lambda n, h: (n, h, 0)),
        ],
        out_specs=pl.BlockSpec((1, _BH, _W), lambda n, h: (n, h, 0)),
        out_shape=jax.ShapeDtypeStruct((_N, _H, _W), jnp.int32),
        interpret=_INTERPRET,
    )(thr, predict, lbl32)

    return out.astype(jnp.int64)


# speedup vs baseline: 1.7873x; 1.7653x over previous
import numpy as np
import jax
import jax.numpy as jnp
from jax.experimental import pallas as pl
from jax.experimental.pallas import tpu as pltpu

_THRESH = 0.6
_MIN_KEPT = 200000
_FACTOR = 8
_IGNORE = -1

_N, _C, _H, _W = 4, 19, 512, 512
_OH, _OW = 64, 64
_NDS = _N * _OH * _OW
_K = min(_NDS, _MIN_KEPT // (_FACTOR * _FACTOR)) - 1
_BH = 128
_NHB = _H // _BH
_DSB = _OH // _NHB

_INTERPRET = False


def _grid_coords(size, out):
    c = (np.arange(out) * (size - 1)).astype(np.float32) / np.float32(out - 1)
    lo = np.floor(c).astype(np.int32)
    hi = np.minimum(lo + 1, size - 1).astype(np.int32)
    frac = (c - lo.astype(np.float32)).astype(np.float32)
    near = np.clip(np.floor(c + 0.5).astype(np.int32), 0, size - 1)
    return lo, hi, frac, near


_H0, _H1, _FH, _IH = _grid_coords(_H, _OH)
_W0, _W1, _FW, _IW = _grid_coords(_W, _OW)

_RW = np.zeros((_NHB, _DSB, _BH), np.float32)
for _i in range(_OH):
    _hb, _il = _i // _DSB, _i % _DSB
    _RW[_hb, _il, _H0[_i] - _BH * _hb] += np.float32(1.0) - _FH[_i]
    _RW[_hb, _il, _H1[_i] - _BH * _hb] += _FH[_i]

_CW = np.zeros((_W, _OW), np.float32)
_CW[_W0, np.arange(_OW)] += np.float32(1.0) - _FW
_CW[_W1, np.arange(_OW)] += _FW

_IMAP = np.zeros(_H, np.int32)
_RVALID = np.zeros(_H, bool)
_IMAP[_H0] = np.arange(_OH)
_RVALID[_H0] = True
_IMAP[_H1] = np.arange(_OH)
_RVALID[_H1] = True
_JMAP = np.zeros(_W, np.int32)
_CVALID = np.zeros(_W, bool)
_JMAP[_W0] = np.arange(_OW)
_CVALID[_W0] = True
_JMAP[_W1] = np.arange(_OW)
_CVALID[_W1] = True


def _main_kernel(x_ref, lbl_ref, l2_ref, rw_ref, cw_ref,
                 pred_ref, ds_ref):
    x = x_ref[0]
    l = lbl_ref[0]
    l2 = l2_ref[0]
    m = x[0]
    for c in range(1, _C):
        m = jnp.maximum(m, x[c])
    s = jnp.zeros_like(m)
    el = jnp.zeros_like(m)
    t2 = jnp.zeros_like(m)
    for c in range(_C):
        e = jnp.exp(x[c] - m)
        s = s + e
        el = jnp.where(l == c, e, el)
        t2 = jnp.where(l2 == c, e, t2)
    pred_ref[0] = el / s
    v = t2 / s
    p = jax.lax.dot_general(
        rw_ref[0], v, (((1,), (0,)), ((), ())),
        preferred_element_type=jnp.float32)
    ds_ref[0, 0] = jax.lax.dot_general(
        p, cw_ref[...], (((1,), (0,)), ((), ())),
        preferred_element_type=jnp.float32)


def _threshold_kernel(ds_ref, out):
    v = jax.lax.bitcast_convert_type(ds_ref[...], jnp.int32)

    def body(_, carry):
        lo_b, hi_b = carry
        mid = lo_b + (hi_b - lo_b) // 2
        cnt = jnp.sum((v <= mid).astype(jnp.int32))
        ge = cnt >= (_K + 1)
        return (jnp.where(ge, lo_b, mid + 1), jnp.where(ge, mid, hi_b))

    lo_b, _hi = jax.lax.fori_loop(
        0, 31, body, (jnp.int32(0), jnp.int32(0x7F7FFFFF)))
    kth = jax.lax.bitcast_convert_type(lo_b, jnp.float32)
    out[0, 0] = jnp.where(kth > _THRESH, kth, jnp.float32(_THRESH))


def _mask_kernel(thr_ref, pred_ref, lbl_ref, out_ref):
    t = thr_ref[0, 0]
    l = lbl_ref[0]
    keep = (l >= 0) & (pred_ref[0] <= t)
    out_ref[0] = jnp.where(keep, l, _IGNORE)


def kernel(predict, target):
    lbl32 = target.astype(jnp.int32)

    lbl_ds = lbl32[:, _IH][:, :, _IW]
    l2 = jnp.where(
        jnp.asarray(_RVALID)[None, :, None]
        & jnp.asarray(_CVALID)[None, None, :],
        lbl_ds[:, _IMAP][:, :, _JMAP], -1)

    pred_map, ds = pl.pallas_call(
        _main_kernel,
        grid=(_N, _NHB),
        in_specs=[
            pl.BlockSpec((1, _C, _BH, _W), lambda n, h: (n, 0, h, 0)),
            pl.BlockSpec((1, _BH, _W), lambda n, h: (n, h, 0)),
            pl.BlockSpec((1, _BH, _W), lambda n, h: (n, h, 0)),
            pl.BlockSpec((1, _DSB, _BH), lambda n, h: (h, 0, 0)),
            pl.BlockSpec((_W, _OW), lambda n, h: (0, 0)),
        ],
        out_specs=[
            pl.BlockSpec((1, _BH, _W), lambda n, h: (n, h, 0)),
            pl.BlockSpec((1, 1, _DSB, _OW), lambda n, h: (n, h, 0, 0)),
        ],
        out_shape=[
            jax.ShapeDtypeStruct((_N, _H, _W), jnp.float32),
            jax.ShapeDtypeStruct((_N, _NHB, _DSB, _OW), jnp.float32),
        ],
        interpret=_INTERPRET,
    )(predict, lbl32, l2, jnp.asarray(_RW), jnp.asarray(_CW))

    thr = pl.pallas_call(
        _threshold_kernel,
        out_shape=jax.ShapeDtypeStruct((1, 1), jnp.float32),
        out_specs=pl.BlockSpec(memory_space=pltpu.SMEM),
        interpret=_INTERPRET,
    )(ds)

    out = pl.pallas_call(
        _mask_kernel,
        grid=(_N, _NHB),
        in_specs=[
            pl.BlockSpec(memory_space=pltpu.SMEM),
            pl.BlockSpec((1, _BH, _W), lambda n, h: (n, h, 0)),
            pl.BlockSpec((1, _BH, _W), lambda n, h: (n, h, 0)),
        ],
        out_specs=pl.BlockSpec((1, _BH, _W), lambda n, h: (n, h, 0)),
        out_shape=jax.ShapeDtypeStruct((_N, _H, _W), jnp.int32),
        interpret=_INTERPRET,
    )(thr, pred_map, lbl32)

    return out.astype(jnp.int64)


# speedup vs baseline: 2.4824x; 1.3889x over previous
import numpy as np
import jax
import jax.numpy as jnp
from jax.experimental import pallas as pl
from jax.experimental.pallas import tpu as pltpu

_THRESH = 0.6
_MIN_KEPT = 200000
_FACTOR = 8
_IGNORE = -1

_N, _C, _H, _W = 4, 19, 512, 512
_OH, _OW = 64, 64
_NDS = _N * _OH * _OW
_K = min(_NDS, _MIN_KEPT // (_FACTOR * _FACTOR)) - 1
_BH = 128
_NHB = _H // _BH
_DSB = _OH // _NHB

_INTERPRET = False


def _grid_coords(size, out):
    c = (np.arange(out) * (size - 1)).astype(np.float32) / np.float32(out - 1)
    lo = np.floor(c).astype(np.int32)
    hi = np.minimum(lo + 1, size - 1).astype(np.int32)
    frac = (c - lo.astype(np.float32)).astype(np.float32)
    near = np.clip(np.floor(c + 0.5).astype(np.int32), 0, size - 1)
    return lo, hi, frac, near


_H0, _H1, _FH, _IH = _grid_coords(_H, _OH)
_W0, _W1, _FW, _IW = _grid_coords(_W, _OW)

_RW = np.zeros((_NHB, _DSB, _BH), np.float32)
for _i in range(_OH):
    _hb, _il = _i // _DSB, _i % _DSB
    _RW[_hb, _il, _H0[_i] - _BH * _hb] += np.float32(1.0) - _FH[_i]
    _RW[_hb, _il, _H1[_i] - _BH * _hb] += _FH[_i]

_CW = np.zeros((_W, _OW), np.float32)
_CW[_W0, np.arange(_OW)] += np.float32(1.0) - _FW
_CW[_W1, np.arange(_OW)] += _FW

_JMAP = np.zeros(_W, np.int32)
_CVALID = np.zeros(_W, bool)
_JMAP[_W0] = np.arange(_OW)
_CVALID[_W0] = True
_JMAP[_W1] = np.arange(_OW)
_CVALID[_W1] = True
_ECOL = np.zeros((_OW, _W), np.float32)
_ECOL[_JMAP[_CVALID], np.nonzero(_CVALID)[0]] = 1.0
_EROW = np.zeros((_NHB, _BH, _OH), np.float32)
for _i in range(_OH):
    _hb = _i // _DSB
    _EROW[_hb, _H0[_i] - _BH * _hb, _i] = 1.0
    _EROW[_hb, _H1[_i] - _BH * _hb, _i] = 1.0


def _main_kernel(x_ref, lbl_ref, lds_ref, erow_ref, ecol_ref, rw_ref, cw_ref,
                 pred_ref, ds_ref):
    x = x_ref[0]
    l = lbl_ref[0]
    colx = jax.lax.dot_general(
        lds_ref[0], ecol_ref[...], (((1,), (0,)), ((), ())),
        preferred_element_type=jnp.float32)
    l2f = jax.lax.dot_general(
        erow_ref[0], colx, (((1,), (0,)), ((), ())),
        preferred_element_type=jnp.float32)
    m = x[0]
    for c in range(1, _C):
        m = jnp.maximum(m, x[c])
    s = jnp.zeros_like(m)
    el = jnp.zeros_like(m)
    t2 = jnp.zeros_like(m)
    for c in range(_C):
        e = jnp.exp(x[c] - m)
        s = s + e
        el = jnp.where(l == c, e, el)
        t2 = jnp.where(l2f == np.float32(c + 1), e, t2)
    pred_ref[0] = el / s
    v = t2 / s
    p = jax.lax.dot_general(
        rw_ref[0], v, (((1,), (0,)), ((), ())),
        preferred_element_type=jnp.float32)
    ds_ref[0, 0] = jax.lax.dot_general(
        p, cw_ref[...], (((1,), (0,)), ((), ())),
        preferred_element_type=jnp.float32)


def _threshold_kernel(ds_ref, out):
    v = jax.lax.bitcast_convert_type(ds_ref[...], jnp.int32)

    def body(_, carry):
        lo_b, hi_b = carry
        mid = lo_b + (hi_b - lo_b) // 2
        cnt = jnp.sum((v <= mid).astype(jnp.int32))
        ge = cnt >= (_K + 1)
        return (jnp.where(ge, lo_b, mid + 1), jnp.where(ge, mid, hi_b))

    lo_b, _hi = jax.lax.fori_loop(
        0, 31, body, (jnp.int32(0), jnp.int32(0x7F7FFFFF)))
    kth = jax.lax.bitcast_convert_type(lo_b, jnp.float32)
    out[0, 0] = jnp.where(kth > _THRESH, kth, jnp.float32(_THRESH))


def _mask_kernel(thr_ref, pred_ref, lbl_ref, out_ref):
    t = thr_ref[0, 0]
    l = lbl_ref[0]
    keep = (l >= 0) & (pred_ref[0] <= t)
    out_ref[0] = jnp.where(keep, l, _IGNORE)


def kernel(predict, target):
    lbl32 = target.astype(jnp.int32)

    lbl_ds = lbl32[:, _IH][:, :, _IW]
    lds_f = (lbl_ds + 1).astype(jnp.float32)

    pred_map, ds = pl.pallas_call(
        _main_kernel,
        grid=(_N, _NHB),
        in_specs=[
            pl.BlockSpec((1, _C, _BH, _W), lambda n, h: (n, 0, h, 0)),
            pl.BlockSpec((1, _BH, _W), lambda n, h: (n, h, 0)),
            pl.BlockSpec((1, _OH, _OW), lambda n, h: (n, 0, 0)),
            pl.BlockSpec((1, _BH, _OH), lambda n, h: (h, 0, 0)),
            pl.BlockSpec((_OW, _W), lambda n, h: (0, 0)),
            pl.BlockSpec((1, _DSB, _BH), lambda n, h: (h, 0, 0)),
            pl.BlockSpec((_W, _OW), lambda n, h: (0, 0)),
        ],
        out_specs=[
            pl.BlockSpec((1, _BH, _W), lambda n, h: (n, h, 0)),
            pl.BlockSpec((1, 1, _DSB, _OW), lambda n, h: (n, h, 0, 0)),
        ],
        out_shape=[
            jax.ShapeDtypeStruct((_N, _H, _W), jnp.float32),
            jax.ShapeDtypeStruct((_N, _NHB, _DSB, _OW), jnp.float32),
        ],
        interpret=_INTERPRET,
    )(predict, lbl32, lds_f, jnp.asarray(_EROW), jnp.asarray(_ECOL),
      jnp.asarray(_RW), jnp.asarray(_CW))

    thr = pl.pallas_call(
        _threshold_kernel,
        out_shape=jax.ShapeDtypeStruct((1, 1), jnp.float32),
        out_specs=pl.BlockSpec(memory_space=pltpu.SMEM),
        interpret=_INTERPRET,
    )(ds)

    out = pl.pallas_call(
        _mask_kernel,
        grid=(_N, _NHB),
        in_specs=[
            pl.BlockSpec(memory_space=pltpu.SMEM),
            pl.BlockSpec((1, _BH, _W), lambda n, h: (n, h, 0)),
            pl.BlockSpec((1, _BH, _W), lambda n, h: (n, h, 0)),
        ],
        out_specs=pl.BlockSpec((1, _BH, _W), lambda n, h: (n, h, 0)),
        out_shape=jax.ShapeDtypeStruct((_N, _H, _W), jnp.int32),
        interpret=_INTERPRET,
    )(thr, pred_map, lbl32)

    return out.astype(jnp.int64)


# speedup vs baseline: 2.6225x; 1.0564x over previous
import numpy as np
import jax
import jax.numpy as jnp
from jax.experimental import pallas as pl
from jax.experimental.pallas import tpu as pltpu

_THRESH = 0.6
_MIN_KEPT = 200000
_FACTOR = 8
_IGNORE = -1

_N, _C, _H, _W = 4, 19, 512, 512
_OH, _OW = 64, 64
_NDS = _N * _OH * _OW
_K = min(_NDS, _MIN_KEPT // (_FACTOR * _FACTOR)) - 1
_BH = 128
_NHB = _H // _BH
_DSB = _OH // _NHB

_INTERPRET = False


def _grid_coords(size, out):
    c = (np.arange(out) * (size - 1)).astype(np.float32) / np.float32(out - 1)
    lo = np.floor(c).astype(np.int32)
    hi = np.minimum(lo + 1, size - 1).astype(np.int32)
    frac = (c - lo.astype(np.float32)).astype(np.float32)
    near = np.clip(np.floor(c + 0.5).astype(np.int32), 0, size - 1)
    return lo, hi, frac, near


_H0, _H1, _FH, _IH = _grid_coords(_H, _OH)
_W0, _W1, _FW, _IW = _grid_coords(_W, _OW)

_RW = np.zeros((_NHB, _DSB, _BH), np.float32)
for _i in range(_OH):
    _hb, _il = _i // _DSB, _i % _DSB
    _RW[_hb, _il, _H0[_i] - _BH * _hb] += np.float32(1.0) - _FH[_i]
    _RW[_hb, _il, _H1[_i] - _BH * _hb] += _FH[_i]

_CW = np.zeros((_W, _OW), np.float32)
_CW[_W0, np.arange(_OW)] += np.float32(1.0) - _FW
_CW[_W1, np.arange(_OW)] += _FW

_JMAP = np.zeros(_W, np.int32)
_CVALID = np.zeros(_W, bool)
_JMAP[_W0] = np.arange(_OW)
_CVALID[_W0] = True
_JMAP[_W1] = np.arange(_OW)
_CVALID[_W1] = True
_ECOL = np.zeros((_OW, _W), np.float32)
_ECOL[_JMAP[_CVALID], np.nonzero(_CVALID)[0]] = 1.0
_EROW = np.zeros((_NHB, _BH, _DSB), np.float32)
for _i in range(_OH):
    _hb = _i // _DSB
    _EROW[_hb, _H0[_i] - _BH * _hb, _i % _DSB] = 1.0
    _EROW[_hb, _H1[_i] - _BH * _hb, _i % _DSB] = 1.0
_NSEL = np.zeros((_NHB, _DSB, _BH), np.float32)
for _i in range(_OH):
    _hb = _i // _DSB
    _NSEL[_hb, _i % _DSB, _IH[_i] - _BH * _hb] = 1.0
_NCOL = np.zeros((_W, _OW), np.float32)
_NCOL[_IW, np.arange(_OW)] = 1.0


def _dot(a, b):
    return jax.lax.dot_general(
        a, b, (((1,), (0,)), ((), ())), preferred_element_type=jnp.float32)


def _main_kernel(x_ref, lbl_ref, nsel_ref, ncol_ref, erow_ref, ecol_ref,
                 rw_ref, cw_ref, pred_ref, ds_ref):
    x = x_ref[0]
    l = lbl_ref[0]
    lblf = (l + 1).astype(jnp.float32)
    ldsb = _dot(_dot(nsel_ref[0], lblf), ncol_ref[...])
    l2f = _dot(erow_ref[0], _dot(ldsb, ecol_ref[...]))
    m = x[0]
    for c in range(1, _C):
        m = jnp.maximum(m, x[c])
    s = jnp.zeros_like(m)
    el = jnp.zeros_like(m)
    t2 = jnp.zeros_like(m)
    for c in range(_C):
        e = jnp.exp(x[c] - m)
        s = s + e
        el = jnp.where(l == c, e, el)
        t2 = jnp.where(l2f == np.float32(c + 1), e, t2)
    pred_ref[0] = el / s
    v = t2 / s
    p = _dot(rw_ref[0], v)
    ds_ref[0, 0] = _dot(p, cw_ref[...])


def _mask_kernel(ds_ref, pred_ref, lbl_ref, out_ref, thr_ref):
    i = pl.program_id(0)

    @pl.when(i == 0)
    def _threshold():
        v = jax.lax.bitcast_convert_type(ds_ref[...], jnp.int32)

        def body(_, carry):
            lo_b, hi_b = carry
            mid = lo_b + (hi_b - lo_b) // 2
            cnt = jnp.sum((v <= mid).astype(jnp.int32))
            ge = cnt >= (_K + 1)
            return (jnp.where(ge, lo_b, mid + 1), jnp.where(ge, mid, hi_b))

        lo_b, _hi = jax.lax.fori_loop(
            0, 31, body, (jnp.int32(0), jnp.int32(0x7F7FFFFF)))
        kth = jax.lax.bitcast_convert_type(lo_b, jnp.float32)
        thr_ref[0] = jnp.where(kth > _THRESH, kth, jnp.float32(_THRESH))

    @pl.when(i > 0)
    def _mask():
        t = thr_ref[0]
        l = lbl_ref[0]
        keep = (l >= 0) & (pred_ref[0] <= t)
        out_ref[0] = jnp.where(keep, l, _IGNORE)


def kernel(predict, target):
    lbl32 = target.astype(jnp.int32)

    pred_map, ds = pl.pallas_call(
        _main_kernel,
        grid=(_N, _NHB),
        in_specs=[
            pl.BlockSpec((1, _C, _BH, _W), lambda n, h: (n, 0, h, 0)),
            pl.BlockSpec((1, _BH, _W), lambda n, h: (n, h, 0)),
            pl.BlockSpec((1, _DSB, _BH), lambda n, h: (h, 0, 0)),
            pl.BlockSpec((_W, _OW), lambda n, h: (0, 0)),
            pl.BlockSpec((1, _BH, _DSB), lambda n, h: (h, 0, 0)),
            pl.BlockSpec((_OW, _W), lambda n, h: (0, 0)),
            pl.BlockSpec((1, _DSB, _BH), lambda n, h: (h, 0, 0)),
            pl.BlockSpec((_W, _OW), lambda n, h: (0, 0)),
        ],
        out_specs=[
            pl.BlockSpec((1, _BH, _W), lambda n, h: (n, h, 0)),
            pl.BlockSpec((1, 1, _DSB, _OW), lambda n, h: (n, h, 0, 0)),
        ],
        out_shape=[
            jax.ShapeDtypeStruct((_N, _H, _W), jnp.float32),
            jax.ShapeDtypeStruct((_N, _NHB, _DSB, _OW), jnp.float32),
        ],
        interpret=_INTERPRET,
    )(predict, lbl32, jnp.asarray(_NSEL), jnp.asarray(_NCOL),
      jnp.asarray(_EROW), jnp.asarray(_ECOL),
      jnp.asarray(_RW), jnp.asarray(_CW))

    nblk = _N * _NHB

    def _blk(i):
        j = jnp.maximum(i - 1, 0)
        return (j // _NHB, j % _NHB, 0)

    out = pl.pallas_call(
        _mask_kernel,
        grid=(nblk + 1,),
        in_specs=[
            pl.BlockSpec((_N, _NHB, _DSB, _OW), lambda i: (0, 0, 0, 0)),
            pl.BlockSpec((1, _BH, _W), _blk),
            pl.BlockSpec((1, _BH, _W), _blk),
        ],
        out_specs=pl.BlockSpec((1, _BH, _W), _blk),
        out_shape=jax.ShapeDtypeStruct((_N, _H, _W), jnp.int32),
        scratch_shapes=[pltpu.SMEM((1,), jnp.float32)],
        interpret=_INTERPRET,
    )(ds, pred_map, lbl32)

    return out.astype(jnp.int64)


# speedup vs baseline: 3.0463x; 1.1616x over previous
import numpy as np
import jax
import jax.numpy as jnp
from jax.experimental import pallas as pl
from jax.experimental.pallas import tpu as pltpu

_THRESH = 0.6
_MIN_KEPT = 200000
_FACTOR = 8
_IGNORE = -1

_N, _C, _H, _W = 4, 19, 512, 512
_OH, _OW = 64, 64
_NDS = _N * _OH * _OW
_K = min(_NDS, _MIN_KEPT // (_FACTOR * _FACTOR)) - 1
_BH = 256
_NHB = _H // _BH
_DSB = _OH // _NHB

_INTERPRET = False


def _grid_coords(size, out):
    c = (np.arange(out) * (size - 1)).astype(np.float32) / np.float32(out - 1)
    lo = np.floor(c).astype(np.int32)
    hi = np.minimum(lo + 1, size - 1).astype(np.int32)
    frac = (c - lo.astype(np.float32)).astype(np.float32)
    near = np.clip(np.floor(c + 0.5).astype(np.int32), 0, size - 1)
    return lo, hi, frac, near


_H0, _H1, _FH, _IH = _grid_coords(_H, _OH)
_W0, _W1, _FW, _IW = _grid_coords(_W, _OW)

_RW = np.zeros((_NHB, _DSB, _BH), np.float32)
for _i in range(_OH):
    _hb, _il = _i // _DSB, _i % _DSB
    _RW[_hb, _il, _H0[_i] - _BH * _hb] += np.float32(1.0) - _FH[_i]
    _RW[_hb, _il, _H1[_i] - _BH * _hb] += _FH[_i]

_CW = np.zeros((_W, _OW), np.float32)
_CW[_W0, np.arange(_OW)] += np.float32(1.0) - _FW
_CW[_W1, np.arange(_OW)] += _FW

_JMAP = np.zeros(_W, np.int32)
_CVALID = np.zeros(_W, bool)
_JMAP[_W0] = np.arange(_OW)
_CVALID[_W0] = True
_JMAP[_W1] = np.arange(_OW)
_CVALID[_W1] = True
_ECOL = np.zeros((_OW, _W), np.float32)
_ECOL[_JMAP[_CVALID], np.nonzero(_CVALID)[0]] = 1.0
_EROW = np.zeros((_NHB, _BH, _DSB), np.float32)
for _i in range(_OH):
    _hb = _i // _DSB
    _EROW[_hb, _H0[_i] - _BH * _hb, _i % _DSB] = 1.0
    _EROW[_hb, _H1[_i] - _BH * _hb, _i % _DSB] = 1.0
_NSEL = np.zeros((_NHB, _DSB, _BH), np.float32)
for _i in range(_OH):
    _hb = _i // _DSB
    _NSEL[_hb, _i % _DSB, _IH[_i] - _BH * _hb] = 1.0
_NCOL = np.zeros((_W, _OW), np.float32)
_NCOL[_IW, np.arange(_OW)] = 1.0


def _dot(a, b):
    return jax.lax.dot_general(
        a, b, (((1,), (0,)), ((), ())), preferred_element_type=jnp.float32)


def _main_kernel(x_ref, lbl_ref, nsel_ref, ncol_ref, erow_ref, ecol_ref,
                 rw_ref, cw_ref, pred_ref, ds_ref):
    x = x_ref[0]
    l = lbl_ref[0]
    lblf = (l + 1).astype(jnp.float32)
    ldsb = _dot(_dot(nsel_ref[0], lblf), ncol_ref[...])
    l2f = _dot(erow_ref[0], _dot(ldsb, ecol_ref[...]))
    m = x[0]
    for c in range(1, _C):
        m = jnp.maximum(m, x[c])
    s = jnp.zeros_like(m)
    el = jnp.zeros_like(m)
    t2 = jnp.zeros_like(m)
    for c in range(_C):
        e = jnp.exp(x[c] - m)
        s = s + e
        el = jnp.where(l == c, e, el)
        t2 = jnp.where(l2f == np.float32(c + 1), e, t2)
    pred_ref[0] = el / s
    v = t2 / s
    p = _dot(rw_ref[0], v)
    ds_ref[0, 0] = _dot(p, cw_ref[...])


def _mask_kernel(ds_ref, pred_ref, lbl_ref, out_ref, thr_ref):
    i = pl.program_id(0)

    @pl.when(i == 0)
    def _threshold():
        v = jax.lax.bitcast_convert_type(ds_ref[...], jnp.int32)

        def body(_, carry):
            lo_b, hi_b = carry
            mid = lo_b + (hi_b - lo_b) // 2
            cnt = jnp.sum((v <= mid).astype(jnp.int32))
            ge = cnt >= (_K + 1)
            return (jnp.where(ge, lo_b, mid + 1), jnp.where(ge, mid, hi_b))

        lo_b, _hi = jax.lax.fori_loop(
            0, 31, body, (jnp.int32(0), jnp.int32(0x7F7FFFFF)))
        kth = jax.lax.bitcast_convert_type(lo_b, jnp.float32)
        thr_ref[0] = jnp.where(kth > _THRESH, kth, jnp.float32(_THRESH))

    @pl.when(i > 0)
    def _mask():
        t = thr_ref[0]
        l = lbl_ref[0]
        keep = (l >= 0) & (pred_ref[0] <= t)
        out_ref[0] = jnp.where(keep, l, _IGNORE)


def kernel(predict, target):
    lbl32 = target.astype(jnp.int32)

    pred_map, ds = pl.pallas_call(
        _main_kernel,
        grid=(_N, _NHB),
        in_specs=[
            pl.BlockSpec((1, _C, _BH, _W), lambda n, h: (n, 0, h, 0)),
            pl.BlockSpec((1, _BH, _W), lambda n, h: (n, h, 0)),
            pl.BlockSpec((1, _DSB, _BH), lambda n, h: (h, 0, 0)),
            pl.BlockSpec((_W, _OW), lambda n, h: (0, 0)),
            pl.BlockSpec((1, _BH, _DSB), lambda n, h: (h, 0, 0)),
            pl.BlockSpec((_OW, _W), lambda n, h: (0, 0)),
            pl.BlockSpec((1, _DSB, _BH), lambda n, h: (h, 0, 0)),
            pl.BlockSpec((_W, _OW), lambda n, h: (0, 0)),
        ],
        out_specs=[
            pl.BlockSpec((1, _BH, _W), lambda n, h: (n, h, 0)),
            pl.BlockSpec((1, 1, _DSB, _OW), lambda n, h: (n, h, 0, 0)),
        ],
        out_shape=[
            jax.ShapeDtypeStruct((_N, _H, _W), jnp.float32),
            jax.ShapeDtypeStruct((_N, _NHB, _DSB, _OW), jnp.float32),
        ],
        interpret=_INTERPRET,
    )(predict, lbl32, jnp.asarray(_NSEL), jnp.asarray(_NCOL),
      jnp.asarray(_EROW), jnp.asarray(_ECOL),
      jnp.asarray(_RW), jnp.asarray(_CW))

    nblk = _N * _NHB

    def _blk(i):
        j = jnp.maximum(i - 1, 0)
        return (j // _NHB, j % _NHB, 0)

    out = pl.pallas_call(
        _mask_kernel,
        grid=(nblk + 1,),
        in_specs=[
            pl.BlockSpec((_N, _NHB, _DSB, _OW), lambda i: (0, 0, 0, 0)),
            pl.BlockSpec((1, _BH, _W), _blk),
            pl.BlockSpec((1, _BH, _W), _blk),
        ],
        out_specs=pl.BlockSpec((1, _BH, _W), _blk),
        out_shape=jax.ShapeDtypeStruct((_N, _H, _W), jnp.int32),
        scratch_shapes=[pltpu.SMEM((1,), jnp.float32)],
        interpret=_INTERPRET,
    )(ds, pred_map, lbl32)

    return out.astype(jnp.int64)


# speedup vs baseline: 3.1894x; 1.0470x over previous
import numpy as np
import jax
import jax.numpy as jnp
from jax.experimental import pallas as pl
from jax.experimental.pallas import tpu as pltpu

_THRESH = 0.6
_MIN_KEPT = 200000
_FACTOR = 8
_IGNORE = -1

_N, _C, _H, _W = 4, 19, 512, 512
_OH, _OW = 64, 64
_NDS = _N * _OH * _OW
_K = min(_NDS, _MIN_KEPT // (_FACTOR * _FACTOR)) - 1
_BH = 512
_NHB = _H // _BH
_DSB = _OH // _NHB

_INTERPRET = False


def _grid_coords(size, out):
    c = (np.arange(out) * (size - 1)).astype(np.float32) / np.float32(out - 1)
    lo = np.floor(c).astype(np.int32)
    hi = np.minimum(lo + 1, size - 1).astype(np.int32)
    frac = (c - lo.astype(np.float32)).astype(np.float32)
    near = np.clip(np.floor(c + 0.5).astype(np.int32), 0, size - 1)
    return lo, hi, frac, near


_H0, _H1, _FH, _IH = _grid_coords(_H, _OH)
_W0, _W1, _FW, _IW = _grid_coords(_W, _OW)

_RW = np.zeros((_NHB, _DSB, _BH), np.float32)
for _i in range(_OH):
    _hb, _il = _i // _DSB, _i % _DSB
    _RW[_hb, _il, _H0[_i] - _BH * _hb] += np.float32(1.0) - _FH[_i]
    _RW[_hb, _il, _H1[_i] - _BH * _hb] += _FH[_i]

_CW = np.zeros((_W, _OW), np.float32)
_CW[_W0, np.arange(_OW)] += np.float32(1.0) - _FW
_CW[_W1, np.arange(_OW)] += _FW

_JMAP = np.zeros(_W, np.int32)
_CVALID = np.zeros(_W, bool)
_JMAP[_W0] = np.arange(_OW)
_CVALID[_W0] = True
_JMAP[_W1] = np.arange(_OW)
_CVALID[_W1] = True
_ECOL = np.zeros((_OW, _W), np.float32)
_ECOL[_JMAP[_CVALID], np.nonzero(_CVALID)[0]] = 1.0
_EROW = np.zeros((_NHB, _BH, _DSB), np.float32)
for _i in range(_OH):
    _hb = _i // _DSB
    _EROW[_hb, _H0[_i] - _BH * _hb, _i % _DSB] = 1.0
    _EROW[_hb, _H1[_i] - _BH * _hb, _i % _DSB] = 1.0
_NSEL = np.zeros((_NHB, _DSB, _BH), np.float32)
for _i in range(_OH):
    _hb = _i // _DSB
    _NSEL[_hb, _i % _DSB, _IH[_i] - _BH * _hb] = 1.0
_NCOL = np.zeros((_W, _OW), np.float32)
_NCOL[_IW, np.arange(_OW)] = 1.0


def _dot(a, b):
    return jax.lax.dot_general(
        a, b, (((1,), (0,)), ((), ())), preferred_element_type=jnp.float32)


def _main_kernel(x_ref, lbl_ref, nsel_ref, ncol_ref, erow_ref, ecol_ref,
                 rw_ref, cw_ref, pred_ref, ds_ref):
    x = x_ref[0]
    l = lbl_ref[0]
    lblf = (l + 1).astype(jnp.float32)
    ldsb = _dot(_dot(nsel_ref[0], lblf), ncol_ref[...])
    l2f = _dot(erow_ref[0], _dot(ldsb, ecol_ref[...]))
    m = x[0]
    for c in range(1, _C):
        m = jnp.maximum(m, x[c])
    s = jnp.zeros_like(m)
    el = jnp.zeros_like(m)
    t2 = jnp.zeros_like(m)
    for c in range(_C):
        e = jnp.exp(x[c] - m)
        s = s + e
        el = jnp.where(l == c, e, el)
        t2 = jnp.where(l2f == np.float32(c + 1), e, t2)
    pred_ref[0] = el / s
    v = t2 / s
    p = _dot(rw_ref[0], v)
    ds_ref[0, 0] = _dot(p, cw_ref[...])


def _mask_kernel(ds_ref, pred_ref, lbl_ref, out_ref, thr_ref):
    i = pl.program_id(0)

    @pl.when(i == 0)
    def _threshold():
        v = jax.lax.bitcast_convert_type(ds_ref[...], jnp.int32)

        def body(_, carry):
            lo_b, hi_b = carry
            mid = lo_b + (hi_b - lo_b) // 2
            cnt = jnp.sum((v <= mid).astype(jnp.int32))
            ge = cnt >= (_K + 1)
            return (jnp.where(ge, lo_b, mid + 1), jnp.where(ge, mid, hi_b))

        lo_b, _hi = jax.lax.fori_loop(
            0, 31, body, (jnp.int32(0), jnp.int32(0x7F7FFFFF)))
        kth = jax.lax.bitcast_convert_type(lo_b, jnp.float32)
        thr_ref[0] = jnp.where(kth > _THRESH, kth, jnp.float32(_THRESH))

    @pl.when(i > 0)
    def _mask():
        t = thr_ref[0]
        l = lbl_ref[0]
        keep = (l >= 0) & (pred_ref[0] <= t)
        out_ref[0] = jnp.where(keep, l, _IGNORE)


def kernel(predict, target):
    lbl32 = target.astype(jnp.int32)

    pred_map, ds = pl.pallas_call(
        _main_kernel,
        grid=(_N, _NHB),
        in_specs=[
            pl.BlockSpec((1, _C, _BH, _W), lambda n, h: (n, 0, h, 0)),
            pl.BlockSpec((1, _BH, _W), lambda n, h: (n, h, 0)),
            pl.BlockSpec((1, _DSB, _BH), lambda n, h: (h, 0, 0)),
            pl.BlockSpec((_W, _OW), lambda n, h: (0, 0)),
            pl.BlockSpec((1, _BH, _DSB), lambda n, h: (h, 0, 0)),
            pl.BlockSpec((_OW, _W), lambda n, h: (0, 0)),
            pl.BlockSpec((1, _DSB, _BH), lambda n, h: (h, 0, 0)),
            pl.BlockSpec((_W, _OW), lambda n, h: (0, 0)),
        ],
        out_specs=[
            pl.BlockSpec((1, _BH, _W), lambda n, h: (n, h, 0)),
            pl.BlockSpec((1, 1, _DSB, _OW), lambda n, h: (n, h, 0, 0)),
        ],
        out_shape=[
            jax.ShapeDtypeStruct((_N, _H, _W), jnp.float32),
            jax.ShapeDtypeStruct((_N, _NHB, _DSB, _OW), jnp.float32),
        ],
        interpret=_INTERPRET,
    )(predict, lbl32, jnp.asarray(_NSEL), jnp.asarray(_NCOL),
      jnp.asarray(_EROW), jnp.asarray(_ECOL),
      jnp.asarray(_RW), jnp.asarray(_CW))

    nblk = _N * _NHB

    def _blk(i):
        j = jnp.maximum(i - 1, 0)
        return (j // _NHB, j % _NHB, 0)

    out = pl.pallas_call(
        _mask_kernel,
        grid=(nblk + 1,),
        in_specs=[
            pl.BlockSpec((_N, _NHB, _DSB, _OW), lambda i: (0, 0, 0, 0)),
            pl.BlockSpec((1, _BH, _W), _blk),
            pl.BlockSpec((1, _BH, _W), _blk),
        ],
        out_specs=pl.BlockSpec((1, _BH, _W), _blk),
        out_shape=jax.ShapeDtypeStruct((_N, _H, _W), jnp.int32),
        scratch_shapes=[pltpu.SMEM((1,), jnp.float32)],
        interpret=_INTERPRET,
    )(ds, pred_map, lbl32)

    return out.astype(jnp.int64)


# speedup vs baseline: 3.5782x; 1.1219x over previous
import numpy as np
import jax
import jax.numpy as jnp
from jax.experimental import pallas as pl
from jax.experimental.pallas import tpu as pltpu

_THRESH = 0.6
_MIN_KEPT = 200000
_FACTOR = 8
_IGNORE = -1

_N, _C, _H, _W = 4, 19, 512, 512
_OH, _OW = 64, 64
_NDS = _N * _OH * _OW
_K = min(_NDS, _MIN_KEPT // (_FACTOR * _FACTOR)) - 1
_BH = 512
_NHB = _H // _BH
_DSB = _OH // _NHB

_INTERPRET = False


def _grid_coords(size, out):
    c = (np.arange(out) * (size - 1)).astype(np.float32) / np.float32(out - 1)
    lo = np.floor(c).astype(np.int32)
    hi = np.minimum(lo + 1, size - 1).astype(np.int32)
    frac = (c - lo.astype(np.float32)).astype(np.float32)
    near = np.clip(np.floor(c + 0.5).astype(np.int32), 0, size - 1)
    return lo, hi, frac, near


_H0, _H1, _FH, _IH = _grid_coords(_H, _OH)
_W0, _W1, _FW, _IW = _grid_coords(_W, _OW)

_RW = np.zeros((_NHB, _DSB, _BH), np.float32)
for _i in range(_OH):
    _hb, _il = _i // _DSB, _i % _DSB
    _RW[_hb, _il, _H0[_i] - _BH * _hb] += np.float32(1.0) - _FH[_i]
    _RW[_hb, _il, _H1[_i] - _BH * _hb] += _FH[_i]

_CW = np.zeros((_W, _OW), np.float32)
_CW[_W0, np.arange(_OW)] += np.float32(1.0) - _FW
_CW[_W1, np.arange(_OW)] += _FW

_JMAP = np.zeros(_W, np.int32)
_CVALID = np.zeros(_W, bool)
_JMAP[_W0] = np.arange(_OW)
_CVALID[_W0] = True
_JMAP[_W1] = np.arange(_OW)
_CVALID[_W1] = True
_ECOL = np.zeros((_OW, _W), np.float32)
_ECOL[_JMAP[_CVALID], np.nonzero(_CVALID)[0]] = 1.0
_EROW = np.zeros((_NHB, _BH, _DSB), np.float32)
for _i in range(_OH):
    _hb = _i // _DSB
    _EROW[_hb, _H0[_i] - _BH * _hb, _i % _DSB] = 1.0
    _EROW[_hb, _H1[_i] - _BH * _hb, _i % _DSB] = 1.0
_NSEL = np.zeros((_NHB, _DSB, _BH), np.float32)
for _i in range(_OH):
    _hb = _i // _DSB
    _NSEL[_hb, _i % _DSB, _IH[_i] - _BH * _hb] = 1.0
_NCOL = np.zeros((_W, _OW), np.float32)
_NCOL[_IW, np.arange(_OW)] = 1.0


def _dot(a, b):
    return jax.lax.dot_general(
        a, b, (((1,), (0,)), ((), ())), preferred_element_type=jnp.float32)


_NBLK = _N * _NHB


def _fused_kernel(x_ref, lbl_ref, nsel_ref, ncol_ref, erow_ref, ecol_ref,
                  rw_ref, cw_ref, out_ref,
                  pred_s, lbl_s, ds_s, thr_s):
    i = pl.program_id(0)

    @pl.when(i < _NBLK)
    def _stream():
        _stream_block(x_ref, lbl_ref, nsel_ref, ncol_ref, erow_ref,
                      ecol_ref, rw_ref, cw_ref, pred_s, lbl_s, ds_s, i)

    @pl.when(i == _NBLK)
    def _threshold():
        v = jax.lax.bitcast_convert_type(ds_s[...], jnp.int32)

        def body(_, carry):
            lo_b, hi_b = carry
            mid = lo_b + (hi_b - lo_b) // 2
            cnt = jnp.sum((v <= mid).astype(jnp.int32))
            ge = cnt >= (_K + 1)
            return (jnp.where(ge, lo_b, mid + 1), jnp.where(ge, mid, hi_b))

        lo_b, _hi = jax.lax.fori_loop(
            0, 31, body, (jnp.int32(0), jnp.int32(0x7F7FFFFF)))
        kth = jax.lax.bitcast_convert_type(lo_b, jnp.float32)
        thr_s[0] = jnp.where(kth > _THRESH, kth, jnp.float32(_THRESH))

    @pl.when(i >= _NBLK)
    def _mask():
        k = i - _NBLK
        t = thr_s[0]
        l = lbl_s[k]
        keep = (l >= 0) & (pred_s[k] <= t)
        out_ref[0] = jnp.where(keep, l, _IGNORE)


def _stream_block(x_ref, lbl_ref, nsel_ref, ncol_ref, erow_ref, ecol_ref,
                  rw_ref, cw_ref, pred_s, lbl_s, ds_s, blk):
    x = x_ref[0]
    l = lbl_ref[0]
    lblf = (l + 1).astype(jnp.float32)
    ldsb = _dot(_dot(nsel_ref[0], lblf), ncol_ref[...])
    l2f = _dot(erow_ref[0], _dot(ldsb, ecol_ref[...]))
    m = x[0]
    for c in range(1, _C):
        m = jnp.maximum(m, x[c])
    s = jnp.zeros_like(m)
    el = jnp.zeros_like(m)
    t2 = jnp.zeros_like(m)
    for c in range(_C):
        e = jnp.exp(x[c] - m)
        s = s + e
        el = jnp.where(l == c, e, el)
        t2 = jnp.where(l2f == np.float32(c + 1), e, t2)
    pred_s[blk] = el / s
    lbl_s[blk] = l
    v = t2 / s
    p = _dot(rw_ref[0], v)
    ds_s[blk] = _dot(p, cw_ref[...])


def kernel(predict, target):
    lbl32 = target.astype(jnp.int32)

    def _in_blk(i):
        j = jnp.minimum(i, _NBLK - 1)
        return (j // _NHB, j % _NHB, 0)

    def _in_blk_x(i):
        j = jnp.minimum(i, _NBLK - 1)
        return (j // _NHB, 0, j % _NHB, 0)

    def _in_blk_h(i):
        return (jnp.minimum(i, _NBLK - 1) % _NHB, 0, 0)

    def _out_blk(i):
        k = jnp.maximum(i - _NBLK, 0)
        return (k // _NHB, k % _NHB, 0)

    out = pl.pallas_call(
        _fused_kernel,
        grid=(2 * _NBLK,),
        in_specs=[
            pl.BlockSpec((1, _C, _BH, _W), _in_blk_x),
            pl.BlockSpec((1, _BH, _W), _in_blk),
            pl.BlockSpec((1, _DSB, _BH), _in_blk_h),
            pl.BlockSpec((_W, _OW), lambda i: (0, 0)),
            pl.BlockSpec((1, _BH, _DSB), _in_blk_h),
            pl.BlockSpec((_OW, _W), lambda i: (0, 0)),
            pl.BlockSpec((1, _DSB, _BH), _in_blk_h),
            pl.BlockSpec((_W, _OW), lambda i: (0, 0)),
        ],
        out_specs=pl.BlockSpec((1, _BH, _W), _out_blk),
        out_shape=jax.ShapeDtypeStruct((_N, _H, _W), jnp.int32),
        scratch_shapes=[
            pltpu.VMEM((_NBLK, _BH, _W), jnp.float32),
            pltpu.VMEM((_NBLK, _BH, _W), jnp.int32),
            pltpu.VMEM((_NBLK, _DSB, _OW), jnp.float32),
            pltpu.SMEM((1,), jnp.float32),
        ],
        interpret=_INTERPRET,
    )(predict, lbl32, jnp.asarray(_NSEL), jnp.asarray(_NCOL),
      jnp.asarray(_EROW), jnp.asarray(_ECOL),
      jnp.asarray(_RW), jnp.asarray(_CW))

    return out.astype(jnp.int64)
